# baseline (device time: 56488 ns/iter reference)
import functools

import jax
import jax.numpy as jnp
from jax import lax
from jax.experimental import pallas as pl
from jax.experimental.pallas import tpu as pltpu

N_DEV = 8

PERM_A = [0, 4, 6, 2, 1, 5, 7, 3]
PERM_B = [0, 1, 5, 4, 2, 3, 7, 6]
MASKS_A = (4, 3, 1)
MASKS_B = (1, 4, 3)


def kernel(x, w_mat):
    m_per, k = x.shape
    _, n_per = w_mat.shape
    h = m_per // 2

    def body(x_ref, w_ref, out_ref, xga_ref, xgb_ref, send_sems, recv_sems):
        my = lax.axis_index("i")
        b0 = jnp.bitwise_and(my, 1)
        b1 = jnp.bitwise_and(jnp.right_shift(my, 1), 1)
        b2 = jnp.bitwise_and(jnp.right_shift(my, 2), 1)
        tA = b2 + 2 * b1 + 4 * jnp.bitwise_xor(b0, b1)
        tB = jnp.bitwise_xor(b0, b1) + 2 * b2 + 4 * b1

        barrier = pltpu.get_barrier_semaphore()
        for msk in (1, 3, 4):
            pl.semaphore_signal(
                barrier, inc=1,
                device_id=(jnp.bitwise_xor(my, msk),),
                device_id_type=pl.DeviceIdType.MESH,
            )
        pl.semaphore_wait(barrier, 3)

        xbf = x_ref[...].astype(jnp.bfloat16)
        xga_ref[tA] = xbf[:h]
        xgb_ref[tB] = xbf[h:]

        def phase(ref, tm, msk, sidx, pidx):
            n = 1 << pidx
            a = jnp.bitwise_and(tm, -n)
            partner = jnp.bitwise_xor(my, msk)
            send = pltpu.make_async_remote_copy(
                src_ref=ref.at[pl.ds(a, n)],
                dst_ref=ref.at[pl.ds(a, n)],
                send_sem=send_sems.at[sidx, pidx],
                recv_sem=recv_sems.at[sidx, pidx],
                device_id=(partner,),
                device_id_type=pl.DeviceIdType.MESH,
            )
            send.start()
            recv = pltpu.make_async_remote_copy(
                src_ref=ref.at[pl.ds(a, n)],
                dst_ref=ref.at[pl.ds(jnp.bitwise_xor(a, n), n)],
                send_sem=send_sems.at[sidx, pidx],
                recv_sem=recv_sems.at[sidx, pidx],
                device_id=(partner,),
                device_id_type=pl.DeviceIdType.MESH,
            )
            return send, recv

        sends = []
        sa0, ra0 = phase(xga_ref, tA, MASKS_A[0], 0, 0)
        sb0, rb0 = phase(xgb_ref, tB, MASKS_B[0], 1, 0)
        ra0.wait_recv()
        sa1, ra1 = phase(xga_ref, tA, MASKS_A[1], 0, 1)
        rb0.wait_recv()
        sb1, rb1 = phase(xgb_ref, tB, MASKS_B[1], 1, 1)
        ra1.wait_recv()
        sa2, ra2 = phase(xga_ref, tA, MASKS_A[2], 0, 2)
        rb1.wait_recv()
        sb2, rb2 = phase(xgb_ref, tB, MASKS_B[2], 1, 2)
        ra2.wait_recv()
        rb2.wait_recv()
        sends += [sa0, sb0, sa1, sb1, sa2, sb2]

        w = w_ref[...].astype(jnp.bfloat16)
        for s in range(N_DEV):
            ya = jnp.dot(xga_ref[PERM_A[s]], w,
                         preferred_element_type=jnp.float32)
            out_ref[pl.ds(s * m_per, h), :] = ya * jax.nn.sigmoid(ya)
            yb = jnp.dot(xgb_ref[PERM_B[s]], w,
                         preferred_element_type=jnp.float32)
            out_ref[pl.ds(s * m_per + h, h), :] = yb * jax.nn.sigmoid(yb)

        for snd in sends:
            snd.wait_send()

        @functools.partial(pl.run_scoped, sem2=pltpu.SemaphoreType.REGULAR)
        def _(sem2):
            for msk in (1, 3, 4):
                pl.semaphore_signal(
                    sem2, inc=1,
                    device_id=(jnp.bitwise_xor(my, msk),),
                    device_id_type=pl.DeviceIdType.MESH,
                )
            pl.semaphore_wait(sem2, 3)

    return pl.pallas_call(
        body,
        out_shape=jax.ShapeDtypeStruct((N_DEV * m_per, n_per), jnp.float32),
        in_specs=[
            pl.BlockSpec(memory_space=pltpu.VMEM),
            pl.BlockSpec(memory_space=pltpu.VMEM),
        ],
        out_specs=pl.BlockSpec(memory_space=pltpu.VMEM),
        scratch_shapes=[
            pltpu.VMEM((N_DEV, m_per // 2, k), jnp.bfloat16),
            pltpu.VMEM((N_DEV, m_per // 2, k), jnp.bfloat16),
            pltpu.SemaphoreType.DMA((2, 3)),
            pltpu.SemaphoreType.DMA((2, 3)),
        ],
        compiler_params=pltpu.CompilerParams(collective_id=0),
    )(x, w_mat)


# device time: 46410 ns/iter; 1.2172x vs baseline; 1.2172x over previous
import functools

import jax
import jax.numpy as jnp
from jax import lax
from jax.experimental import pallas as pl
from jax.experimental.pallas import tpu as pltpu

N_DEV = 8

PERM_A = [0, 4, 6, 2, 1, 5, 7, 3]
PERM_B = [0, 1, 5, 4, 2, 3, 7, 6]
MASKS_A = (4, 3, 1)
MASKS_B = (1, 4, 3)


def kernel(x, w_mat):
    m_per, k = x.shape
    _, n_per = w_mat.shape
    h = m_per // 2

    def body(x_ref, w_ref, out_ref, xga_ref, xgb_ref, send_sems, recv_sems):
        my = lax.axis_index("i")
        b0 = jnp.bitwise_and(my, 1)
        b1 = jnp.bitwise_and(jnp.right_shift(my, 1), 1)
        b2 = jnp.bitwise_and(jnp.right_shift(my, 2), 1)
        tA = b2 + 2 * b1 + 4 * jnp.bitwise_xor(b0, b1)
        tB = jnp.bitwise_xor(b0, b1) + 2 * b2 + 4 * b1

        barrier = pltpu.get_barrier_semaphore()
        for msk in (1, 3, 4):
            pl.semaphore_signal(
                barrier, inc=1,
                device_id=(jnp.bitwise_xor(my, msk),),
                device_id_type=pl.DeviceIdType.MESH,
            )
        pl.semaphore_wait(barrier, 3)

        xbf = x_ref[...].astype(jnp.bfloat16)
        xga_ref[tA] = xbf[:h]
        xgb_ref[tB] = xbf[h:]

        def snd(ref, sidx, i, start, n, msk):
            r = pltpu.make_async_remote_copy(
                src_ref=ref.at[pl.ds(start, n)],
                dst_ref=ref.at[pl.ds(start, n)],
                send_sem=send_sems.at[sidx, i],
                recv_sem=recv_sems.at[sidx, i],
                device_id=(jnp.bitwise_xor(my, msk),),
                device_id_type=pl.DeviceIdType.MESH,
            )
            r.start()
            return r

        def rcv(ref, sidx, i, start, n):
            r = pltpu.make_async_remote_copy(
                src_ref=ref.at[pl.ds(start, n)],
                dst_ref=ref.at[pl.ds(start, n)],
                send_sem=send_sems.at[sidx, i],
                recv_sem=recv_sems.at[sidx, i],
                device_id=(my,),
                device_id_type=pl.DeviceIdType.MESH,
            )
            r.wait_recv()

        X = jnp.bitwise_xor
        sends = []
        for ref, t, (m0, m1, m2), P in (
            (xga_ref, tA, MASKS_A, 0),
            (xgb_ref, tB, MASKS_B, 1),
        ):
            sends.append(snd(ref, P, 0, t, 1, m0))
            sends.append(snd(ref, P, 1, t, 1, m1))
            sends.append(snd(ref, P, 3, t, 1, m2))
        for ref, t, (m0, m1, m2), P in (
            (xga_ref, tA, MASKS_A, 0),
            (xgb_ref, tB, MASKS_B, 1),
        ):
            rcv(ref, P, 0, X(t, 1), 1)
            sends.append(snd(ref, P, 2, X(t, 1), 1, m1))
            sends.append(snd(ref, P, 4, X(t, 1), 1, m2))
        for ref, t, (m0, m1, m2), P in (
            (xga_ref, tA, MASKS_A, 0),
            (xgb_ref, tB, MASKS_B, 1),
        ):
            rcv(ref, P, 1, X(t, 2), 1)
            rcv(ref, P, 2, X(t, 3), 1)
            a2 = X(jnp.bitwise_and(t, -2), 2)
            sends.append(snd(ref, P, 5, a2, 2, m2))

        w = w_ref[...].astype(jnp.bfloat16)
        rcv(xga_ref, 0, 3, X(tA, 4), 1)
        rcv(xga_ref, 0, 4, X(tA, 5), 1)
        rcv(xga_ref, 0, 5, X(jnp.bitwise_and(tA, -2), 6), 2)
        for s in range(N_DEV):
            ya = jnp.dot(xga_ref[PERM_A[s]], w,
                         preferred_element_type=jnp.float32)
            out_ref[pl.ds(s * m_per, h), :] = ya * jax.nn.sigmoid(ya)
        rcv(xgb_ref, 1, 3, X(tB, 4), 1)
        rcv(xgb_ref, 1, 4, X(tB, 5), 1)
        rcv(xgb_ref, 1, 5, X(jnp.bitwise_and(tB, -2), 6), 2)
        for s in range(N_DEV):
            yb = jnp.dot(xgb_ref[PERM_B[s]], w,
                         preferred_element_type=jnp.float32)
            out_ref[pl.ds(s * m_per + h, h), :] = yb * jax.nn.sigmoid(yb)

        for s_ in sends:
            s_.wait_send()

        @functools.partial(pl.run_scoped, sem2=pltpu.SemaphoreType.REGULAR)
        def _(sem2):
            for msk in (1, 3, 4):
                pl.semaphore_signal(
                    sem2, inc=1,
                    device_id=(jnp.bitwise_xor(my, msk),),
                    device_id_type=pl.DeviceIdType.MESH,
                )
            pl.semaphore_wait(sem2, 3)

    return pl.pallas_call(
        body,
        out_shape=jax.ShapeDtypeStruct((N_DEV * m_per, n_per), jnp.float32),
        in_specs=[
            pl.BlockSpec(memory_space=pltpu.VMEM),
            pl.BlockSpec(memory_space=pltpu.VMEM),
        ],
        out_specs=pl.BlockSpec(memory_space=pltpu.VMEM),
        scratch_shapes=[
            pltpu.VMEM((N_DEV, m_per // 2, k), jnp.bfloat16),
            pltpu.VMEM((N_DEV, m_per // 2, k), jnp.bfloat16),
            pltpu.SemaphoreType.DMA((2, 6)),
            pltpu.SemaphoreType.DMA((2, 6)),
        ],
        compiler_params=pltpu.CompilerParams(collective_id=0),
    )(x, w_mat)


# device time: 43829 ns/iter; 1.2888x vs baseline; 1.0589x over previous
import functools

import jax
import jax.numpy as jnp
from jax import lax
from jax.experimental import pallas as pl
from jax.experimental.pallas import tpu as pltpu

N_DEV = 8

PIPES = (
    ((4, 3, 1), [0, 4, 6, 2, 1, 5, 7, 3]),
    ((3, 1, 4), [0, 2, 3, 1, 4, 6, 7, 5]),
    ((1, 4, 3), [0, 1, 5, 4, 2, 3, 7, 6]),
)
SPLITS = ((0, 96), (96, 80), (176, 80))


def kernel(x, w_mat):
    m_per, k = x.shape
    _, n_per = w_mat.shape

    def body(x_ref, w_ref, out_ref, xg0_ref, xg1_ref, xg2_ref,
             send_sems, recv_sems):
        my = lax.axis_index("i")
        b0 = jnp.bitwise_and(my, 1)
        b1 = jnp.bitwise_and(jnp.right_shift(my, 1), 1)
        b2 = jnp.bitwise_and(jnp.right_shift(my, 2), 1)
        b01 = jnp.bitwise_xor(b0, b1)
        refs = [xg0_ref, xg1_ref, xg2_ref]
        ts = [b2 + 2 * b1 + 4 * b01, b1 + 2 * b01 + 4 * b2,
              b01 + 2 * b2 + 4 * b1]

        barrier = pltpu.get_barrier_semaphore()
        for msk in (1, 3, 4):
            pl.semaphore_signal(
                barrier, inc=1,
                device_id=(jnp.bitwise_xor(my, msk),),
                device_id_type=pl.DeviceIdType.MESH,
            )
        pl.semaphore_wait(barrier, 3)

        xbf = x_ref[...].astype(jnp.bfloat16)
        for P in range(3):
            r0, nr = SPLITS[P]
            refs[P][ts[P]] = xbf[r0:r0 + nr]

        def snd(P, i, start, n, msk):
            r = pltpu.make_async_remote_copy(
                src_ref=refs[P].at[pl.ds(start, n)],
                dst_ref=refs[P].at[pl.ds(start, n)],
                send_sem=send_sems.at[P, i],
                recv_sem=recv_sems.at[P, i],
                device_id=(jnp.bitwise_xor(my, msk),),
                device_id_type=pl.DeviceIdType.MESH,
            )
            r.start()
            return r

        def rcv(P, i, start, n):
            r = pltpu.make_async_remote_copy(
                src_ref=refs[P].at[pl.ds(start, n)],
                dst_ref=refs[P].at[pl.ds(start, n)],
                send_sem=send_sems.at[P, i],
                recv_sem=recv_sems.at[P, i],
                device_id=(my,),
                device_id_type=pl.DeviceIdType.MESH,
            )
            r.wait_recv()

        X = jnp.bitwise_xor
        sends = []
        for P, ((m0, m1, m2), _) in enumerate(PIPES):
            sends.append(snd(P, 0, ts[P], 1, m0))
            sends.append(snd(P, 1, ts[P], 1, m1))
            sends.append(snd(P, 3, ts[P], 1, m2))
        for P, ((m0, m1, m2), _) in enumerate(PIPES):
            rcv(P, 0, X(ts[P], 1), 1)
            sends.append(snd(P, 2, X(ts[P], 1), 1, m1))
            sends.append(snd(P, 4, X(ts[P], 1), 1, m2))
        for P, ((m0, m1, m2), _) in enumerate(PIPES):
            rcv(P, 1, X(ts[P], 2), 1)
            rcv(P, 2, X(ts[P], 3), 1)
            a2 = X(jnp.bitwise_and(ts[P], -2), 2)
            sends.append(snd(P, 5, a2, 2, m2))

        w = w_ref[...].astype(jnp.bfloat16)
        for P, (_, perm) in enumerate(PIPES):
            rcv(P, 3, X(ts[P], 4), 1)
            rcv(P, 4, X(ts[P], 5), 1)
            rcv(P, 5, X(jnp.bitwise_and(ts[P], -2), 6), 2)
            r0, nr = SPLITS[P]
            for s in range(N_DEV):
                y = jnp.dot(refs[P][perm[s]], w,
                            preferred_element_type=jnp.float32)
                out_ref[pl.ds(s * m_per + r0, nr), :] = (
                    y * jax.nn.sigmoid(y))

        for s_ in sends:
            s_.wait_send()

        @functools.partial(pl.run_scoped, sem2=pltpu.SemaphoreType.REGULAR)
        def _(sem2):
            for msk in (1, 3, 4):
                pl.semaphore_signal(
                    sem2, inc=1,
                    device_id=(jnp.bitwise_xor(my, msk),),
                    device_id_type=pl.DeviceIdType.MESH,
                )
            pl.semaphore_wait(sem2, 3)

    return pl.pallas_call(
        body,
        out_shape=jax.ShapeDtypeStruct((N_DEV * m_per, n_per), jnp.float32),
        in_specs=[
            pl.BlockSpec(memory_space=pltpu.VMEM),
            pl.BlockSpec(memory_space=pltpu.VMEM),
        ],
        out_specs=pl.BlockSpec(memory_space=pltpu.VMEM),
        scratch_shapes=[
            pltpu.VMEM((N_DEV, SPLITS[0][1], k), jnp.bfloat16),
            pltpu.VMEM((N_DEV, SPLITS[1][1], k), jnp.bfloat16),
            pltpu.VMEM((N_DEV, SPLITS[2][1], k), jnp.bfloat16),
            pltpu.SemaphoreType.DMA((3, 6)),
            pltpu.SemaphoreType.DMA((3, 6)),
        ],
        compiler_params=pltpu.CompilerParams(collective_id=0),
    )(x, w_mat)
